# Initial kernel scaffold; baseline (speedup 1.0000x reference)
import jax
import jax.numpy as jnp
from jax.experimental import pallas as pl


def kernel(x, Wg, bg, W1, b1, W2, b2):
    raise NotImplementedError("write your pallas kernel here")


# speedup vs baseline: 1.0582x; 1.0582x over previous
import jax
import jax.numpy as jnp
from jax import lax
from jax.experimental import pallas as pl
from jax.experimental.pallas import tpu as pltpu

T = 128
HIDDEN = 1024
E = 64
TOPK = 2


def _moe_body(x_ref, Wg_ref, bg_ref, W1_ref, b1_ref, W2_ref, b2_ref,
              out_ref, gate_ref, wmat_ref):
    e = pl.program_id(0)

    @pl.when(e == 0)
    def _gate():
        x = x_ref[...]
        logits = jnp.dot(x, Wg_ref[...], preferred_element_type=jnp.float32)
        logits = logits + bg_ref[...]
        m = jnp.max(logits, axis=1, keepdims=True)
        p = jnp.exp(logits - m)
        gate = p / jnp.sum(p, axis=1, keepdims=True)
        gate_ref[...] = gate

        iota_e = lax.broadcasted_iota(jnp.int32, (T, E), 1)
        m1 = jnp.max(gate, axis=1, keepdims=True)
        a1 = jnp.min(jnp.where(gate == m1, iota_e, E), axis=1, keepdims=True)
        sel1 = iota_e == a1
        gate2 = jnp.where(sel1, -1.0, gate)
        m2 = jnp.max(gate2, axis=1, keepdims=True)
        a2 = jnp.min(jnp.where(gate2 == m2, iota_e, E), axis=1, keepdims=True)
        sel2 = iota_e == a2
        wmat_ref[...] = jnp.where(sel1, m1, 0.0) + jnp.where(sel2, m2, 0.0)
        out_ref[...] = jnp.zeros_like(out_ref)

    onehot = (lax.broadcasted_iota(jnp.int32, (E, 1), 0) == e).astype(jnp.float32)
    col = jnp.dot(wmat_ref[...], onehot, preferred_element_type=jnp.float32)

    @pl.when(jnp.sum(col) > 0.0)
    def _expert():
        h = jnp.dot(x_ref[...], W1_ref[0], preferred_element_type=jnp.float32)
        h = jnp.maximum(h + b1_ref[0], 0.0)
        y = jnp.dot(h, W2_ref[0], preferred_element_type=jnp.float32)
        y = y + b2_ref[0]
        out_ref[...] += col * y


def kernel(x, Wg, bg, W1, b1, W2, b2):
    bg2 = bg.reshape(1, E)
    b1 = b1.reshape(E, 1, HIDDEN)
    b2 = b2.reshape(E, 1, HIDDEN)
    out, gate = pl.pallas_call(
        _moe_body,
        grid=(E,),
        in_specs=[
            pl.BlockSpec((T, HIDDEN), lambda e: (0, 0)),
            pl.BlockSpec((HIDDEN, E), lambda e: (0, 0)),
            pl.BlockSpec((1, E), lambda e: (0, 0)),
            pl.BlockSpec((1, HIDDEN, HIDDEN), lambda e: (e, 0, 0)),
            pl.BlockSpec((1, 1, HIDDEN), lambda e: (e, 0, 0)),
            pl.BlockSpec((1, HIDDEN, HIDDEN), lambda e: (e, 0, 0)),
            pl.BlockSpec((1, 1, HIDDEN), lambda e: (e, 0, 0)),
        ],
        out_specs=[
            pl.BlockSpec((T, HIDDEN), lambda e: (0, 0)),
            pl.BlockSpec((T, E), lambda e: (0, 0)),
        ],
        out_shape=[
            jax.ShapeDtypeStruct((T, HIDDEN), jnp.float32),
            jax.ShapeDtypeStruct((T, E), jnp.float32),
        ],
        scratch_shapes=[pltpu.VMEM((T, E), jnp.float32)],
    )(x, Wg, bg2, W1, b1, W2, b2)
    return (out, gate)
